# baseline (speedup 1.0000x reference)
import functools

import jax
import jax.numpy as jnp
from jax import lax
from jax.experimental import pallas as pl
from jax.experimental.pallas import tpu as pltpu
from jax.experimental.pallas import tpu_sc as plsc

BOHR = 0.52917721067
DAMPING = 0.7

N = 50000
E = 1600000
NC, NS, L = 2, 16, 16
NW = NC * NS
EPW = E // NW
C = 400
NCHUNK = EPW // C
CV = C // L
P = 150016
PS = P // NS
OUT3 = 3 * N


def _rsqrt(x):
    i = plsc.bitcast(x, jnp.int32)
    i = jnp.int32(0x5F3759DF) - lax.shift_right_logical(i, 1)
    y = plsc.bitcast(i, jnp.float32)
    xh = x * jnp.float32(0.5)
    for _ in range(2):
        y = y * (jnp.float32(1.5) - xh * y * y)
    return y


def _field_body(src_h, dst_h, dist_h, vx_h, vy_h, vz_h, ch_h, pol_h, out_h,
                ch_v, pol_v, bufs, zb, accum, sems):
    cid = lax.axis_index("c")
    sid = lax.axis_index("s")
    wid = sid * NC + cid

    tcp1 = pltpu.async_copy(ch_h, ch_v, sems[0][0])
    tcp2 = pltpu.async_copy(pol_h, pol_v, sems[0][0])

    zeros16 = jnp.zeros((L,), jnp.float32)

    def zb_body(i, _):
        zb[pl.ds(i * L, L)] = zeros16
        return 0

    lax.fori_loop(0, 2048 // L, zb_body, 0)

    for k in range(4):
        pltpu.sync_copy(zb, accum.at[pl.ds(sid * PS + k * 2048, 2048)])
    pltpu.sync_copy(zb.at[pl.ds(0, 1184)],
                    accum.at[pl.ds(sid * PS + 8192, 1184)])

    tcp1.wait()
    tcp2.wait()
    plsc.subcore_barrier()

    mb2 = jnp.float32(-BOHR * BOHR)
    mdamp = jnp.float32(-DAMPING)
    one = jnp.float32(1.0)

    def fire_in(b, c):
        (src_v, dst_v, dist_v, vx_v, vy_v, vz_v, *_), (semin, _) = \
            bufs[b], sems[b]
        eb = wid * EPW + c * C
        pltpu.async_copy(src_h.at[pl.ds(eb, C)], src_v, semin)
        pltpu.async_copy(dst_h.at[pl.ds(eb, C)], dst_v, semin)
        pltpu.async_copy(dist_h.at[pl.ds(eb, C)], dist_v, semin)
        pltpu.async_copy(vx_h.at[pl.ds(eb, C)], vx_v, semin)
        pltpu.async_copy(vy_h.at[pl.ds(eb, C)], vy_v, semin)
        pltpu.async_copy(vz_h.at[pl.ds(eb, C)], vz_v, semin)

    def wait_in(b):
        (src_v, dst_v, dist_v, vx_v, vy_v, vz_v, *_), (semin, _) = \
            bufs[b], sems[b]
        for v in (src_v, dst_v, dist_v, vx_v, vy_v, vz_v):
            pltpu.make_async_copy(src_h.at[pl.ds(0, C)], v, semin).wait()

    def fire_sc(b):
        (*_, e_all, ix_all), (_, semsc) = bufs[b], sems[b]
        pltpu.async_copy(e_all, accum.at[ix_all], semsc, add=True)

    def wait_sc(b):
        (*_, e_all, ix_all), (_, semsc) = bufs[b], sems[b]
        pltpu.make_async_copy(e_all, accum.at[ix_all], semsc).wait()

    def compute(b):
        (src_v, dst_v, dist_v, vx_v, vy_v, vz_v,
         e_all, ix_all) = bufs[b]

        def vreg_body(i, _):
            o = i * L
            s = src_v[pl.ds(o, L)]
            dd = dst_v[pl.ds(o, L)]
            dist = dist_v[pl.ds(o, L)]
            q = plsc.load_gather(ch_v, [dd])
            ps_ = plsc.load_gather(pol_v, [s])
            pd_ = plsc.load_gather(pol_v, [dd])
            g = ps_ * pd_
            sg = g * _rsqrt(g)
            d3 = dist * dist * dist
            inv3 = one / d3
            u15 = _rsqrt(sg * inv3)
            damp = one - jnp.exp(mdamp * u15)
            f = mb2 * q * damp * inv3
            i3 = s * 3
            e_all[pl.ds(o, L)] = f * vx_v[pl.ds(o, L)]
            e_all[pl.ds(C + o, L)] = f * vy_v[pl.ds(o, L)]
            e_all[pl.ds(2 * C + o, L)] = f * vz_v[pl.ds(o, L)]
            ix_all[pl.ds(o, L)] = i3
            ix_all[pl.ds(C + o, L)] = i3 + 1
            ix_all[pl.ds(2 * C + o, L)] = i3 + 2
            return 0

        lax.fori_loop(0, CV, vreg_body, 0, unroll=25)

    fire_in(0, 0)

    def pipe_body(gc, _):
        for b in (0, 1):
            c = 2 * gc + b
            fire_in(1 - b, c + 1)
            wait_in(b)

            @pl.when(gc > 0)
            def _():
                wait_sc(b)

            compute(b)
            fire_sc(b)
        return 0

    lax.fori_loop(0, (NCHUNK - 1) // 2, pipe_body, 0)

    wait_in(0)
    wait_sc(0)
    compute(0)
    fire_sc(0)
    wait_sc(1)
    wait_sc(0)

    plsc.subcore_barrier()
    pltpu.sync_copy(accum.at[pl.ds(sid * PS, PS)],
                    out_h.at[pl.ds(cid * P + sid * PS, PS)])


def _chunk_bufs():
    return (
        pltpu.VMEM((C,), jnp.int32),
        pltpu.VMEM((C,), jnp.int32),
        pltpu.VMEM((C,), jnp.float32),
        pltpu.VMEM((C,), jnp.float32),
        pltpu.VMEM((C,), jnp.float32),
        pltpu.VMEM((C,), jnp.float32),
        pltpu.VMEM((3 * C,), jnp.float32),
        pltpu.VMEM((3 * C,), jnp.int32),
    )


_sc_field = functools.partial(
    pl.kernel,
    out_type=jax.ShapeDtypeStruct((2 * P,), jnp.float32),
    mesh=plsc.VectorSubcoreMesh(
        core_axis_name="c", subcore_axis_name="s",
        num_cores=NC, num_subcores=NS),
    compiler_params=pltpu.CompilerParams(
        needs_layout_passes=False, use_tc_tiling_on_sc=False),
    scratch_types=[
        pltpu.VMEM((N,), jnp.float32),
        pltpu.VMEM((N,), jnp.float32),
        (_chunk_bufs(), _chunk_bufs()),
        pltpu.VMEM((2048,), jnp.float32),
        pltpu.VMEM_SHARED((P,), jnp.float32),
        ((pltpu.SemaphoreType.DMA, pltpu.SemaphoreType.DMA),
         (pltpu.SemaphoreType.DMA, pltpu.SemaphoreType.DMA)),
    ],
)(_field_body)


def _add_body(a_ref, o_ref):
    o_ref[...] = a_ref[pl.ds(0, P)] + a_ref[pl.ds(P, P)]


def kernel(species, edge_src, edge_dst, distances, vec, charges,
           polarisability):
    del species
    partials = _sc_field(edge_src, edge_dst, distances,
                         vec[:, 0], vec[:, 1], vec[:, 2],
                         charges, polarisability)
    summed = pl.pallas_call(
        _add_body,
        out_shape=jax.ShapeDtypeStruct((P,), jnp.float32),
    )(partials)
    return summed[:OUT3]


# speedup vs baseline: 1.3526x; 1.3526x over previous
import functools

import jax
import jax.numpy as jnp
from jax import lax
from jax.experimental import pallas as pl
from jax.experimental.pallas import tpu as pltpu
from jax.experimental.pallas import tpu_sc as plsc

BOHR = 0.52917721067
DAMPING = 0.7

N = 50000
E = 1600000
NC, NS, L = 2, 16, 16
NW = NC * NS
EPW = E // NW
C = 400
NCHUNK = EPW // C
CV = C // L
P = 150016
PS = P // NS
OUT3 = 3 * N


def _rsqrt(x):
    i = plsc.bitcast(x, jnp.int32)
    i = jnp.int32(0x5F3759DF) - lax.shift_right_logical(i, 1)
    y = plsc.bitcast(i, jnp.float32)
    xh = x * jnp.float32(0.5)
    for _ in range(2):
        y = y * (jnp.float32(1.5) - xh * y * y)
    return y


def _field_body(src_h, dst_h, dist_h, vx_h, vy_h, vz_h, ch_h, pol_h, out_h,
                ch_v, pol_v, bufs, zb, accum, sems):
    cid = lax.axis_index("c")
    sid = lax.axis_index("s")
    wid = sid * NC + cid

    tcp1 = pltpu.async_copy(ch_h, ch_v, sems[0][0])
    tcp2 = pltpu.async_copy(pol_h, pol_v, sems[0][0])

    zeros16 = jnp.zeros((L,), jnp.float32)

    def zb_body(i, _):
        zb[pl.ds(i * L, L)] = zeros16
        return 0

    lax.fori_loop(0, 2048 // L, zb_body, 0)

    for k in range(4):
        pltpu.sync_copy(zb, accum.at[pl.ds(sid * PS + k * 2048, 2048)])
    pltpu.sync_copy(zb.at[pl.ds(0, 1184)],
                    accum.at[pl.ds(sid * PS + 8192, 1184)])

    tcp1.wait()
    tcp2.wait()
    plsc.subcore_barrier()

    mb2 = jnp.float32(-BOHR * BOHR)
    mdamp = jnp.float32(-DAMPING)
    one = jnp.float32(1.0)

    def fire_in(b, c):
        (src_v, dst_v, dist_v, vx_v, vy_v, vz_v, *_), (semin, _) = \
            bufs[b], sems[b]
        eb = wid * EPW + c * C
        pltpu.async_copy(src_h.at[pl.ds(eb, C)], src_v, semin)
        pltpu.async_copy(dst_h.at[pl.ds(eb, C)], dst_v, semin)
        pltpu.async_copy(dist_h.at[pl.ds(eb, C)], dist_v, semin)
        pltpu.async_copy(vx_h.at[pl.ds(eb, C)], vx_v, semin)
        pltpu.async_copy(vy_h.at[pl.ds(eb, C)], vy_v, semin)
        pltpu.async_copy(vz_h.at[pl.ds(eb, C)], vz_v, semin)

    def wait_in(b):
        (src_v, dst_v, dist_v, vx_v, vy_v, vz_v, *_), (semin, _) = \
            bufs[b], sems[b]
        for v in (src_v, dst_v, dist_v, vx_v, vy_v, vz_v):
            pltpu.make_async_copy(src_h.at[pl.ds(0, C)], v, semin).wait()

    def fire_sc(b):
        (*_, e_all, ix_all), (_, semsc) = bufs[b], sems[b]
        pltpu.async_copy(e_all, accum.at[ix_all], semsc, add=True)

    def wait_sc(b):
        (*_, e_all, ix_all), (_, semsc) = bufs[b], sems[b]
        pltpu.make_async_copy(e_all, accum.at[ix_all], semsc).wait()

    def compute(b):
        (src_v, dst_v, dist_v, vx_v, vy_v, vz_v,
         e_all, ix_all) = bufs[b]

        def vreg_body(i, _):
            o = i * L
            s = src_v[pl.ds(o, L)]
            dd = dst_v[pl.ds(o, L)]
            dist = dist_v[pl.ds(o, L)]
            q = plsc.load_gather(ch_v, [dd])
            ps_ = plsc.load_gather(pol_v, [s])
            pd_ = plsc.load_gather(pol_v, [dd])
            g = ps_ * pd_
            sg = g * _rsqrt(g)
            d3 = dist * dist * dist
            inv3 = one / d3
            u15 = _rsqrt(sg * inv3)
            damp = one - jnp.exp(mdamp * u15)
            f = mb2 * q * damp * inv3
            i3 = s * 3
            e_all[pl.ds(o, L)] = f * vx_v[pl.ds(o, L)]
            e_all[pl.ds(C + o, L)] = f * vy_v[pl.ds(o, L)]
            e_all[pl.ds(2 * C + o, L)] = f * vz_v[pl.ds(o, L)]
            ix_all[pl.ds(o, L)] = i3
            ix_all[pl.ds(C + o, L)] = i3 + 1
            ix_all[pl.ds(2 * C + o, L)] = i3 + 2
            return 0

        lax.fori_loop(0, CV, vreg_body, 0, unroll=6)

    fire_in(0, 0)

    def pipe_body(gc, _):
        for b in (0, 1):
            c = 2 * gc + b
            fire_in(1 - b, c + 1)
            wait_in(b)

            @pl.when(gc > 0)
            def _():
                wait_sc(b)

            compute(b)
            fire_sc(b)
        return 0

    lax.fori_loop(0, (NCHUNK - 1) // 2, pipe_body, 0)

    wait_in(0)
    wait_sc(0)
    compute(0)
    fire_sc(0)
    wait_sc(1)
    wait_sc(0)

    plsc.subcore_barrier()
    pltpu.sync_copy(accum.at[pl.ds(sid * PS, PS)],
                    out_h.at[pl.ds(cid * P + sid * PS, PS)])


def _chunk_bufs():
    return (
        pltpu.VMEM((C,), jnp.int32),
        pltpu.VMEM((C,), jnp.int32),
        pltpu.VMEM((C,), jnp.float32),
        pltpu.VMEM((C,), jnp.float32),
        pltpu.VMEM((C,), jnp.float32),
        pltpu.VMEM((C,), jnp.float32),
        pltpu.VMEM((3 * C,), jnp.float32),
        pltpu.VMEM((3 * C,), jnp.int32),
    )


_sc_field = functools.partial(
    pl.kernel,
    out_type=jax.ShapeDtypeStruct((2 * P,), jnp.float32),
    mesh=plsc.VectorSubcoreMesh(
        core_axis_name="c", subcore_axis_name="s",
        num_cores=NC, num_subcores=NS),
    compiler_params=pltpu.CompilerParams(
        needs_layout_passes=False, use_tc_tiling_on_sc=False),
    scratch_types=[
        pltpu.VMEM((N,), jnp.float32),
        pltpu.VMEM((N,), jnp.float32),
        (_chunk_bufs(), _chunk_bufs()),
        pltpu.VMEM((2048,), jnp.float32),
        pltpu.VMEM_SHARED((P,), jnp.float32),
        ((pltpu.SemaphoreType.DMA, pltpu.SemaphoreType.DMA),
         (pltpu.SemaphoreType.DMA, pltpu.SemaphoreType.DMA)),
    ],
)(_field_body)


def _add_body(a_ref, o_ref):
    o_ref[...] = a_ref[pl.ds(0, P)] + a_ref[pl.ds(P, P)]


def kernel(species, edge_src, edge_dst, distances, vec, charges,
           polarisability):
    del species
    partials = _sc_field(edge_src, edge_dst, distances,
                         vec[:, 0], vec[:, 1], vec[:, 2],
                         charges, polarisability)
    summed = pl.pallas_call(
        _add_body,
        out_shape=jax.ShapeDtypeStruct((P,), jnp.float32),
    )(partials)
    return summed[:OUT3]
